# baseline (speedup 1.0000x reference)
import functools

import jax
import jax.numpy as jnp
from jax import lax
from jax.experimental import pallas as pl
from jax.experimental.pallas import tpu as pltpu
from jax.experimental.pallas import tpu_sc as plsc

_NC = 2
_TOTAL = 16 * 3 * 512 * 512
_PER_C = _TOTAL // _NC
_CHUNK = 524288
_NBUF = 3
_NCH = _PER_C // _CHUNK


@functools.partial(
    pl.kernel,
    out_type=jax.ShapeDtypeStruct((_TOTAL // _CHUNK, _CHUNK), jnp.float32),
    mesh=plsc.ScalarSubcoreMesh(axis_name="c", num_cores=_NC),
    scratch_types=[
        pltpu.MemorySpace.VMEM_SHARED((_NBUF, _CHUNK), jnp.float32),
        pltpu.SemaphoreType.DMA((_NBUF,)),
        pltpu.SemaphoreType.DMA((_NBUF,)),
    ],
)
def _sc_copy(x_hbm, out_hbm, buf, isem, osem):
    cid = lax.axis_index("c")
    base = cid * _NCH

    def in_copy(i, b):
        return pltpu.async_copy(
            x_hbm.at[pl.ds(base + i, 1)], buf.at[pl.ds(b, 1)], isem.at[b])

    def out_copy(i, b):
        return pltpu.async_copy(
            buf.at[pl.ds(b, 1)], out_hbm.at[pl.ds(base + i, 1)], osem.at[b])

    ins, outs = {}, {}
    for i in range(_NBUF):
        ins[i] = in_copy(i, i)
    for i in range(_NCH):
        b = i % _NBUF
        ins[i].wait()
        outs[i] = out_copy(i, b)
        j = i + _NBUF
        if j < _NCH:
            outs[i].wait()
            ins[j] = in_copy(j, b)
    for i in range(max(_NCH - _NBUF, 0), _NCH):
        outs[i].wait()


def kernel(x):
    flat = x.reshape(_TOTAL // _CHUNK, _CHUNK)
    out = _sc_copy(flat)
    return out.reshape(x.shape)
